# Initial kernel scaffold; baseline (speedup 1.0000x reference)
import jax
import jax.numpy as jnp
from jax.experimental import pallas as pl


def kernel(x, edge_index, W1, b1, W2, b2):
    raise NotImplementedError("write your pallas kernel here")


# speedup vs baseline: 23.5329x; 23.5329x over previous
import functools

import jax
import jax.numpy as jnp
from jax import lax
from jax.experimental import pallas as pl
from jax.experimental.pallas import tpu as pltpu
from jax.experimental.pallas import tpu_sc as plsc

N = 10000
D = 128
H = 128
C = 16
E = 320000
EPS = 1e-5

NC = 2
NS = 16
K = 128
NPAD = 10240
ET = 10240
EP = NC * NS * ET
CH = ET // K
RPT = NPAD // NS
RZ = 128

_mesh = functools.partial(
    plsc.VectorSubcoreMesh,
    core_axis_name="c", subcore_axis_name="s",
    num_cores=NC, num_subcores=NS)


def _make_deg():
    @functools.partial(
        pl.kernel,
        out_type=jax.ShapeDtypeStruct((NC, NPAD), jnp.float32),
        mesh=_mesh(),
        scratch_types=[
            pltpu.VMEM((K,), jnp.int32),
            pltpu.VMEM((K,), jnp.float32),
            pltpu.VMEM((RPT,), jnp.float32),
            pltpu.VMEM_SHARED((NPAD,), jnp.float32),
        ],
    )
    def deg_kernel(dst_hbm, out_hbm, dst_v, ones_v, z_v, deg_sh):
        cid = lax.axis_index("c")
        sid = lax.axis_index("s")
        wid = cid * NS + sid
        for j in range(K // 16):
            ones_v[pl.ds(j * 16, 16)] = jnp.ones((16,), jnp.float32)
        for j in range(RPT // 16):
            z_v[pl.ds(j * 16, 16)] = jnp.zeros((16,), jnp.float32)
        pltpu.sync_copy(z_v, deg_sh.at[pl.ds(sid * RPT, RPT)])
        plsc.subcore_barrier()

        def step(k, carry):
            pltpu.sync_copy(dst_hbm.at[pl.ds(wid * ET + k * K, K)], dst_v)
            pltpu.sync_copy(ones_v, deg_sh.at[dst_v], add=True)
            return carry
        lax.fori_loop(0, CH, step, 0)
        plsc.subcore_barrier()

        pltpu.sync_copy(deg_sh.at[pl.ds(sid * RPT, RPT)],
                        out_hbm.at[cid, pl.ds(sid * RPT, RPT)])

    return deg_kernel


def _make_agg(F):
    @functools.partial(
        pl.kernel,
        out_type=jax.ShapeDtypeStruct((NC, NPAD, F), jnp.float32),
        mesh=_mesh(),
        compiler_params=pltpu.CompilerParams(
            use_tc_tiling_on_sc=(F % 128 == 0)),
        scratch_types=[
            pltpu.VMEM((K,), jnp.int32),
            pltpu.VMEM((K,), jnp.int32),
            pltpu.VMEM((K,), jnp.int32),
            pltpu.VMEM((K,), jnp.int32),
            pltpu.VMEM((2, K, F), jnp.float32),
            pltpu.VMEM_SHARED((NPAD, F), jnp.float32),
            pltpu.SemaphoreType.DMA,
            pltpu.SemaphoreType.DMA,
        ],
    )
    def agg_kernel(h_hbm, src_hbm, dst_hbm, out_hbm,
                   idx0_v, idx1_v, dst0_v, dst1_v, rows_v, acc_sh,
                   sem0, sem1):
        cid = lax.axis_index("c")
        sid = lax.axis_index("s")
        base = (cid * NS + sid) * ET

        def zrow(i, carry):
            for j in range(F // 16):
                rows_v[0, i, pl.ds(j * 16, 16)] = jnp.zeros((16,),
                                                            jnp.float32)
            return carry
        lax.fori_loop(0, K, zrow, 0)
        for j in range(RPT // RZ):
            pltpu.sync_copy(rows_v.at[0],
                            acc_sh.at[pl.ds(sid * RPT + j * RZ, RZ)])
        plsc.subcore_barrier()

        sems = (sem0, sem1)
        idxs = (idx0_v, idx1_v)
        dsts = (dst0_v, dst1_v)
        pltpu.sync_copy(src_hbm.at[pl.ds(base, K)], idx0_v)
        pltpu.sync_copy(dst_hbm.at[pl.ds(base, K)], dst0_v)
        pltpu.async_copy(h_hbm.at[idx0_v], rows_v.at[0], sem0)

        def step(i, carry):
            for b in range(2):
                c = 2 * i + b
                nxt = jnp.minimum(c + 1, CH - 1)
                pltpu.sync_copy(src_hbm.at[pl.ds(base + nxt * K, K)],
                                idxs[1 - b])
                pltpu.sync_copy(dst_hbm.at[pl.ds(base + nxt * K, K)],
                                dsts[1 - b])
                pltpu.async_copy(h_hbm.at[idxs[1 - b]],
                                 rows_v.at[1 - b], sems[1 - b])
                pltpu.make_async_copy(h_hbm.at[idxs[b]],
                                      rows_v.at[b], sems[b]).wait()
                pltpu.sync_copy(rows_v.at[b], acc_sh.at[dsts[b]], add=True)
            return carry
        lax.fori_loop(0, CH // 2, step, 0)
        pltpu.make_async_copy(h_hbm.at[idx0_v], rows_v.at[0], sem0).wait()
        plsc.subcore_barrier()

        for j in range(RPT // RZ):
            r0 = sid * RPT + j * RZ
            pltpu.sync_copy(acc_sh.at[pl.ds(r0, RZ)],
                            out_hbm.at[cid, pl.ds(r0, RZ)])

    return agg_kernel


_make_deg = functools.lru_cache(None)(_make_deg)
_make_agg = functools.lru_cache(None)(_make_agg)


def _t1_body(x_ref, w_ref, dc_ref, h_ref, dinv_ref):
    x = x_ref[...]
    mu = jnp.mean(x, axis=0, keepdims=True)
    xc = x - mu
    var = jnp.mean(xc * xc, axis=0, keepdims=True)
    xn = xc * lax.rsqrt(var + EPS)
    alpha = jnp.mean(jnp.abs(xn), axis=1, keepdims=True)
    sx = jnp.sign(xn).astype(jnp.bfloat16)
    w = w_ref[...]
    beta = jnp.mean(jnp.abs(w))
    sw = jnp.sign(w).astype(jnp.bfloat16)
    m = jnp.dot(sx, sw, preferred_element_type=jnp.float32)
    dc = dc_ref[...]
    deg = dc[0, :N] + dc[1, :N] + 1.0
    dinv = lax.rsqrt(deg)
    dinv_ref[...] = dinv
    h_ref[...] = m * (alpha * beta * dinv)


def _t2_body(p_ref, h_ref, dinv_ref, b1_ref, w2_ref, o_ref):
    dinv = dinv_ref[...]
    p = p_ref[...]
    agg = p[0, :N] + p[1, :N] + h_ref[...]
    out1 = agg * dinv + b1_ref[...]
    alpha = jnp.mean(jnp.abs(out1), axis=1, keepdims=True)
    s = jnp.sign(out1).astype(jnp.bfloat16)
    w2 = w2_ref[...]
    beta = jnp.mean(jnp.abs(w2))
    sw = jnp.sign(w2).astype(jnp.bfloat16)
    m = jnp.dot(s, sw, preferred_element_type=jnp.float32)
    o_ref[...] = m * (alpha * beta * dinv)


def _t3_body(q_ref, h2_ref, dinv_ref, b2_ref, o_ref):
    q = q_ref[...]
    z = (q[0, :N] + q[1, :N] + h2_ref[...]) * dinv_ref[...] + b2_ref[...]
    t = z - jnp.max(z, axis=1, keepdims=True)
    o_ref[...] = t - jnp.log(jnp.sum(jnp.exp(t), axis=1, keepdims=True))


_t1_call = pl.pallas_call(
    _t1_body,
    out_shape=(jax.ShapeDtypeStruct((N, H), jnp.float32),
               jax.ShapeDtypeStruct((N, 1), jnp.float32)))

_t2_call = pl.pallas_call(
    _t2_body,
    out_shape=jax.ShapeDtypeStruct((N, C), jnp.float32))

_t3_call = pl.pallas_call(
    _t3_body,
    out_shape=jax.ShapeDtypeStruct((N, C), jnp.float32))


def kernel(x, edge_index, W1, b1, W2, b2):
    pad = jnp.arange(EP - E, dtype=jnp.int32)
    src = jnp.concatenate([edge_index[0], pad % N])
    dst = jnp.concatenate([edge_index[1], N + pad % (NPAD - N)])
    degp = _make_deg()(dst)
    h1p, dinv = _t1_call(x, W1, degp.reshape(NC, NPAD, 1))
    p = _make_agg(H)(h1p, src, dst)
    h2p = _t2_call(p, h1p, dinv, b1.reshape(1, H), W2)
    q = _make_agg(C)(h2p, src, dst)
    return _t3_call(q, h2p, dinv, b2.reshape(1, C))


# speedup vs baseline: 38.1419x; 1.6208x over previous
import functools

import jax
import jax.numpy as jnp
from jax import lax
from jax.experimental import pallas as pl
from jax.experimental.pallas import tpu as pltpu
from jax.experimental.pallas import tpu_sc as plsc

N = 10000
D = 128
H = 128
C = 16
E = 320000
EPS = 1e-5

NC = 2
NS = 16
K = 128
NPAD = 10240
ET = 10240
EP = NC * NS * ET
CH = ET // K
RPT = NPAD // NS
RZ = 128

_mesh = functools.partial(
    plsc.VectorSubcoreMesh,
    core_axis_name="c", subcore_axis_name="s",
    num_cores=NC, num_subcores=NS)


def _extract_dst(comb_v, c, out_v):
    for j in range(K // 16):
        v = comb_v[c, pl.ds(j * 16, 16)]
        out_v[pl.ds(j * 16, 16)] = jnp.bitwise_and(v, (1 << 14) - 1)


def _extract_src(comb_v, c, out_v):
    for j in range(K // 16):
        v = comb_v[c, pl.ds(j * 16, 16)]
        out_v[pl.ds(j * 16, 16)] = lax.shift_right_logical(v, 14)


def _make_deg():
    @functools.partial(
        pl.kernel,
        out_type=jax.ShapeDtypeStruct((NC, NPAD), jnp.float32),
        mesh=_mesh(),
        scratch_types=[
            pltpu.VMEM((CH, K), jnp.int32),
            pltpu.VMEM((K,), jnp.int32),
            pltpu.VMEM((K,), jnp.float32),
            pltpu.VMEM((RPT,), jnp.float32),
            pltpu.VMEM_SHARED((NPAD,), jnp.float32),
        ],
    )
    def deg_kernel(comb_hbm, out_hbm, comb_v, dst_v, ones_v, z_v, deg_sh):
        cid = lax.axis_index("c")
        sid = lax.axis_index("s")
        wid = cid * NS + sid
        pltpu.sync_copy(comb_hbm.at[pl.ds(wid * CH, CH)], comb_v)
        for j in range(K // 16):
            ones_v[pl.ds(j * 16, 16)] = jnp.ones((16,), jnp.float32)
        for j in range(RPT // 16):
            z_v[pl.ds(j * 16, 16)] = jnp.zeros((16,), jnp.float32)
        pltpu.sync_copy(z_v, deg_sh.at[pl.ds(sid * RPT, RPT)])
        plsc.subcore_barrier()

        def step(c, carry):
            _extract_dst(comb_v, c, dst_v)
            pltpu.sync_copy(ones_v, deg_sh.at[dst_v], add=True)
            return carry
        lax.fori_loop(0, CH, step, 0)
        plsc.subcore_barrier()

        pltpu.sync_copy(deg_sh.at[pl.ds(sid * RPT, RPT)],
                        out_hbm.at[cid, pl.ds(sid * RPT, RPT)])

    return deg_kernel


def _make_agg(F, NBUF):
    @functools.partial(
        pl.kernel,
        out_type=jax.ShapeDtypeStruct((NC, NPAD, F), jnp.float32),
        mesh=_mesh(),
        compiler_params=pltpu.CompilerParams(
            use_tc_tiling_on_sc=(F % 128 == 0)),
        scratch_types=[
            pltpu.VMEM((CH, K), jnp.int32),
            pltpu.VMEM((NBUF, K), jnp.int32),
            pltpu.VMEM((K,), jnp.int32),
            pltpu.VMEM((NBUF, K, F), jnp.float32),
            pltpu.VMEM_SHARED((NPAD, F), jnp.float32),
        ] + [pltpu.SemaphoreType.DMA] * NBUF,
    )
    def agg_kernel(h_hbm, comb_hbm, out_hbm,
                   comb_v, idx_v, dst_v, rows_v, acc_sh, *sems):
        cid = lax.axis_index("c")
        sid = lax.axis_index("s")
        wid = cid * NS + sid
        pltpu.sync_copy(comb_hbm.at[pl.ds(wid * CH, CH)], comb_v)

        def zrow(i, carry):
            for j in range(F // 16):
                rows_v[0, i, pl.ds(j * 16, 16)] = jnp.zeros((16,),
                                                            jnp.float32)
            return carry
        lax.fori_loop(0, K, zrow, 0)
        for j in range(RPT // RZ):
            pltpu.sync_copy(rows_v.at[0],
                            acc_sh.at[pl.ds(sid * RPT + j * RZ, RZ)])
        plsc.subcore_barrier()

        def fire(c, b):
            for j in range(K // 16):
                v = comb_v[c, pl.ds(j * 16, 16)]
                idx_v[b, pl.ds(j * 16, 16)] = lax.shift_right_logical(v, 14)
            pltpu.async_copy(h_hbm.at[idx_v.at[b]], rows_v.at[b], sems[b])

        for b in range(NBUF):
            fire(b, b)

        def step(i, carry):
            for b in range(NBUF):
                c = i * NBUF + b
                pltpu.make_async_copy(h_hbm.at[idx_v.at[b]],
                                      rows_v.at[b], sems[b]).wait()
                _extract_dst(comb_v, c, dst_v)
                pltpu.sync_copy(rows_v.at[b], acc_sh.at[dst_v], add=True)

                @pl.when(c + NBUF < CH)
                def _():
                    fire(c + NBUF, b)
            return carry
        lax.fori_loop(0, CH // NBUF, step, 0)
        plsc.subcore_barrier()

        for j in range(RPT // RZ):
            r0 = sid * RPT + j * RZ
            pltpu.sync_copy(acc_sh.at[pl.ds(r0, RZ)],
                            out_hbm.at[cid, pl.ds(r0, RZ)])

    return agg_kernel


_make_deg = functools.lru_cache(None)(_make_deg)
_make_agg = functools.lru_cache(None)(_make_agg)


def _t1_body(x_ref, w_ref, dc_ref, h_ref, dinv_ref):
    x = x_ref[...]
    mu = jnp.mean(x, axis=0, keepdims=True)
    xc = x - mu
    var = jnp.mean(xc * xc, axis=0, keepdims=True)
    xn = xc * lax.rsqrt(var + EPS)
    alpha = jnp.mean(jnp.abs(xn), axis=1, keepdims=True)
    sx = jnp.sign(xn).astype(jnp.bfloat16)
    w = w_ref[...]
    beta = jnp.mean(jnp.abs(w))
    sw = jnp.sign(w).astype(jnp.bfloat16)
    m = jnp.dot(sx, sw, preferred_element_type=jnp.float32)
    dc = dc_ref[...]
    deg = dc[0, :N] + dc[1, :N] + 1.0
    dinv = lax.rsqrt(deg)
    dinv_ref[...] = dinv
    h_ref[...] = m * (alpha * beta * dinv)


def _t2_body(p_ref, h_ref, dinv_ref, b1_ref, w2_ref, o_ref):
    dinv = dinv_ref[...]
    p = p_ref[...]
    agg = p[0, :N] + p[1, :N] + h_ref[...]
    out1 = agg * dinv + b1_ref[...]
    alpha = jnp.mean(jnp.abs(out1), axis=1, keepdims=True)
    s = jnp.sign(out1).astype(jnp.bfloat16)
    w2 = w2_ref[...]
    beta = jnp.mean(jnp.abs(w2))
    sw = jnp.sign(w2).astype(jnp.bfloat16)
    m = jnp.dot(s, sw, preferred_element_type=jnp.float32)
    o_ref[...] = m * (alpha * beta * dinv)


def _t3_body(q_ref, h2_ref, dinv_ref, b2_ref, o_ref):
    q = q_ref[...]
    z = (q[0, :N] + q[1, :N] + h2_ref[...]) * dinv_ref[...] + b2_ref[...]
    t = z - jnp.max(z, axis=1, keepdims=True)
    o_ref[...] = t - jnp.log(jnp.sum(jnp.exp(t), axis=1, keepdims=True))


_t1_call = pl.pallas_call(
    _t1_body,
    out_shape=(jax.ShapeDtypeStruct((N, H), jnp.float32),
               jax.ShapeDtypeStruct((N, 1), jnp.float32)))

_t2_call = pl.pallas_call(
    _t2_body,
    out_shape=jax.ShapeDtypeStruct((N, C), jnp.float32))

_t3_call = pl.pallas_call(
    _t3_body,
    out_shape=jax.ShapeDtypeStruct((N, C), jnp.float32))


def kernel(x, edge_index, W1, b1, W2, b2):
    pad = jnp.arange(EP - E, dtype=jnp.int32)
    src = jnp.concatenate([edge_index[0], pad % N])
    dst = jnp.concatenate([edge_index[1], N + pad % (NPAD - N)])
    comb = jnp.bitwise_or(jnp.left_shift(src, 14), dst).reshape(EP // K, K)
    degp = _make_deg()(comb)
    h1p, dinv = _t1_call(x, W1, degp.reshape(NC, NPAD, 1))
    p = _make_agg(H, 2)(h1p, comb)
    h2p = _t2_call(p, h1p, dinv, b1.reshape(1, H), W2)
    q = _make_agg(C, 8)(h2p, comb)
    return _t3_call(q, h2p, dinv, b2.reshape(1, C))
